# Initial kernel scaffold; baseline (speedup 1.0000x reference)
import jax
import jax.numpy as jnp
from jax.experimental import pallas as pl


def kernel(ego_states, neighbor_states, tW1, tb1, tW2, tb2, tW3, tb3, oW1, ob1, oW2, ob2, oW3, ob3):
    raise NotImplementedError("write your pallas kernel here")


# speedup vs baseline: 1.6642x; 1.6642x over previous
import jax
import jax.numpy as jnp
from jax.experimental import pallas as pl

_T = 10
_O = 10
_NSD = 16
_EXP = 16
_GED = 32
_S = _T + _O

_BM = 512


def _body(ns_ref, ego_ref, w1n_ref, w1e_ref, b1_ref, w2_ref, b2_ref,
          w3_ref, b3_ref, out_ref):
    bm = ego_ref.shape[0]
    ns = ns_ref[...]
    ego = ego_ref[...]

    a1 = jnp.dot(ns, w1n_ref[...], preferred_element_type=jnp.float32)
    e1 = jnp.dot(ego, w1e_ref[...], preferred_element_type=jnp.float32)
    e1x = jnp.broadcast_to(e1[:, None, :], (bm, _S, 128)).reshape(bm * _S, 128)
    s = a1 + e1x + b1_ref[...]
    h1 = jnp.where(s > 0, s, jnp.exp(jnp.minimum(s, 0.0)) - 1.0)

    p2 = jnp.dot(h1, w2_ref[...], preferred_element_type=jnp.float32) + b2_ref[...]
    h2 = jnp.where(p2 > 0, p2, jnp.exp(jnp.minimum(p2, 0.0)) - 1.0)
    out_all = jnp.dot(h2, w3_ref[...], preferred_element_type=jnp.float32) + b3_ref[...]

    nanmask = jnp.isnan(s)
    m64 = jnp.logical_or(nanmask[:, :64], nanmask[:, 64:])
    feat = jnp.where(m64, -jnp.inf, out_all)

    f3 = feat.reshape(bm, _S, 64)
    tmax = jnp.max(f3[:, :_T, :], axis=1)
    omax = jnp.max(f3[:, _T:, :], axis=1)
    tglob = tmax[:, :_GED]
    tglob = jnp.where(jnp.isinf(tglob), jnp.float32(-2.0), tglob)
    oglob = omax[:, _GED:]
    out_ref[...] = jnp.concatenate([tglob, oglob], axis=1)


def kernel(ego_states, neighbor_states, tW1, tb1, tW2, tb2, tW3, tb3,
           oW1, ob1, oW2, ob2, oW3, ob3):
    B = ego_states.shape[0]
    ns_flat = neighbor_states.reshape(B * _S, _NSD)

    w1n = jnp.concatenate([tW1[:_NSD], oW1[:_NSD]], axis=1)
    w1e = jnp.concatenate([tW1[_NSD:], oW1[_NSD:]], axis=1)
    b1 = jnp.concatenate([tb1, ob1])[None, :]
    z2 = jnp.zeros_like(tW2)
    w2 = jnp.concatenate([
        jnp.concatenate([tW2, z2], axis=1),
        jnp.concatenate([z2, oW2], axis=1)], axis=0)
    b2 = jnp.concatenate([tb2, ob2])[None, :]
    z3 = jnp.zeros_like(tW3)
    w3 = jnp.concatenate([
        jnp.concatenate([tW3, z3], axis=1),
        jnp.concatenate([z3, oW3], axis=1)], axis=0)
    b3 = jnp.concatenate([tb3, ob3])[None, :]

    grid = (B // _BM,)
    return pl.pallas_call(
        _body,
        grid=grid,
        in_specs=[
            pl.BlockSpec((_BM * _S, _NSD), lambda i: (i, 0)),
            pl.BlockSpec((_BM, _EXP), lambda i: (i, 0)),
            pl.BlockSpec((_NSD, 128), lambda i: (0, 0)),
            pl.BlockSpec((_EXP, 128), lambda i: (0, 0)),
            pl.BlockSpec((1, 128), lambda i: (0, 0)),
            pl.BlockSpec((128, 64), lambda i: (0, 0)),
            pl.BlockSpec((1, 64), lambda i: (0, 0)),
            pl.BlockSpec((64, 64), lambda i: (0, 0)),
            pl.BlockSpec((1, 64), lambda i: (0, 0)),
        ],
        out_specs=pl.BlockSpec((_BM, 2 * _GED), lambda i: (i, 0)),
        out_shape=jax.ShapeDtypeStruct((B, 2 * _GED), jnp.float32),
    )(ns_flat, ego_states, w1n, w1e, b1, w2, b2, w3, b3)
